# baseline (speedup 1.0000x reference)
import dataclasses

import jax
import jax.numpy as jnp
from jax import lax
from jax.experimental import pallas as pl
from jax.experimental.pallas import tpu as pltpu
from jax.experimental.pallas import tpu_sc as plsc

N_USER = 25000
N_ITEM = 25000
N = N_USER + N_ITEM
E = 800000
EMB = 64
HALF = EMB // 2
BATCH = 1024
N_NEG = 4096
NOUT = BATCH + BATCH + N_NEG

NS = 16
EPW = E // NS
SB = 4096
NSB = 12
TAIL = EPW - NSB * SB
K = 128
STG = SB + 2 * K
ACC_ROWS = NOUT
ZROWS = 48
ZREP = (ACC_ROWS // NS) // ZROWS
OUT_PER_SUB = (NOUT // K) // NS
OFF_ROW = 0
OFF_COL = E
OFF_VB = 2 * E
OFF_NEG = 3 * E
OFF_NEED = 3 * E + N


def _half_pass(ego_ref, out_u, out_p, out_n, coff, aux, acc, remap, needbuf, eb0, eb1, scol, sval, srow, frow,
               ridx, gbuf, gbuf2, frow2, zbuf, ebuf, abuf, sem0, sem1,
               fsem, ssem):
    s = lax.axis_index("s")
    zero16i = jnp.zeros((16,), jnp.int32)
    zero16f = jnp.zeros((16,), jnp.float32)
    neg16i = jnp.full((16,), -1, jnp.int32)
    iota16 = lax.iota(jnp.int32, 16)
    ebase = s * EPW
    pltpu.async_copy(aux.at[pl.ds(OFF_ROW + ebase, SB)], eb0.at[0], sem0)
    pltpu.async_copy(aux.at[pl.ds(OFF_COL + ebase, SB)], eb0.at[1], sem0)
    pltpu.async_copy(aux.at[pl.ds(OFF_VB + ebase, SB)], eb0.at[2], sem0)

    hneg = pltpu.async_copy(aux.at[pl.ds(OFF_NEG, N)], remap, sem1)
    pltpu.sync_copy(aux.at[pl.ds(OFF_NEED, NOUT)], needbuf)

    hneg.wait()

    @pl.loop(0, NOUT // 16)
    def _(i):
        idx16 = needbuf[pl.ds(i * 16, 16)]
        plsc.store_scatter(remap, [idx16], i * 16 + iota16)

    @pl.loop(0, ZROWS)
    def _(i):
        zbuf[i, pl.ds(0, 16)] = zero16f
        zbuf[i, pl.ds(16, 16)] = zero16f

    @pl.loop(0, ZREP)
    def _(j):
        pltpu.sync_copy(
            zbuf, acc.at[pl.ds(s * (ACC_ROWS // NS) + j * ZROWS, ZROWS)])

    plsc.subcore_barrier()

    def load_sb(b, buf, sem, size):
        base = pl.multiple_of(ebase + b * SB, 8)
        hr = pltpu.async_copy(aux.at[pl.ds(OFF_ROW + base, size)],
                              buf.at[0].at[pl.ds(0, size)], sem)
        hc = pltpu.async_copy(aux.at[pl.ds(OFF_COL + base, size)],
                              buf.at[1].at[pl.ds(0, size)], sem)
        hv = pltpu.async_copy(aux.at[pl.ds(OFF_VB + base, size)],
                              buf.at[2].at[pl.ds(0, size)], sem)
        return (hr, hc, hv)

    def fire_issue(fb, frowx, gbufx):
        fb = pl.multiple_of(fb, 8)
        for i in range(8):
            frowx[pl.ds(i * 16, 16)] = srow[pl.ds(fb + i * 16, 16)]
        return pltpu.async_copy(ego_ref.at[scol.at[pl.ds(fb, K)]], gbufx,
                                fsem)

    def fire_scale(fb, gbufx):
        fb = pl.multiple_of(fb, 8)

        @pl.loop(0, K, step=16)
        def _(c0):
            vals = sval[pl.ds(fb + c0, 16)]
            for i in range(16):
                vi = vals.at[jnp.full((16,), i, jnp.int32)].get(
                    mode="promise_in_bounds")
                k = c0 + i
                gbufx[k, pl.ds(0, 16)] = gbufx[k, pl.ds(0, 16)] * vi
                gbufx[k, pl.ds(16, 16)] = gbufx[k, pl.ds(16, 16)] * vi

    def fire(fb):
        h = fire_issue(fb, frow, gbuf)
        h.wait()
        fire_scale(fb, gbuf)
        pltpu.sync_copy(gbuf, acc.at[frow], add=True)

    def process_sb(buf, pos, ngroups):
        def grp(g, pos):
            r16 = buf[0, pl.ds(g * 16, 16)]
            m16 = plsc.load_gather(remap, [r16])
            pred = m16 >= 0
            c16 = buf[1, pl.ds(g * 16, 16)]
            v16 = plsc.bitcast(buf[2, pl.ds(g * 16, 16)], jnp.float32)
            plsc.store_compressed(scol.at[pl.ds(pos, 16)], c16, mask=pred)
            plsc.store_compressed(sval.at[pl.ds(pos, 16)], v16, mask=pred)
            plsc.store_compressed(srow.at[pl.ds(pos, 16)], m16, mask=pred)
            cnt = plsc.all_reduce_population_count(pred)
            return pos + cnt[0]

        pos = lax.fori_loop(0, ngroups, grp, pos, unroll=2)

        def fire_cond(st):
            j, p = st
            return j + K <= p

        def fire_body(st):
            j, p = st
            two = j + 2 * K <= p
            ha = fire_issue(j, frow, gbuf)

            @pl.when(two)
            def _():
                fire_issue(j + K, frow2, gbuf2)

            ha.wait()
            fire_scale(j, gbuf)
            hsa = pltpu.async_copy(gbuf, acc.at[frow], ssem, add=True)

            @pl.when(two)
            def _():
                fbb = pl.multiple_of(j + K, 8)
                pltpu.make_async_copy(
                    ego_ref.at[scol.at[pl.ds(fbb, K)]], gbuf2, fsem).wait()
                fire_scale(j + K, gbuf2)
                pltpu.async_copy(gbuf2, acc.at[frow2], ssem, add=True)

            hsa.wait()

            @pl.when(two)
            def _():
                pltpu.make_async_copy(gbuf2, acc.at[frow2], ssem).wait()

            return (j + K + jnp.where(two, K, 0).astype(jnp.int32), p)

        fb, pos = lax.while_loop(fire_cond, fire_body, (jnp.int32(0), pos))
        for i in range(8):
            scol[pl.ds(i * 16, 16)] = scol[pl.ds(fb + i * 16, 16)]
            sval[pl.ds(i * 16, 16)] = sval[pl.ds(fb + i * 16, 16)]
            srow[pl.ds(i * 16, 16)] = srow[pl.ds(fb + i * 16, 16)]
        return pos - fb

    def pair(q, pos):
        b = q * 2
        pltpu.make_async_copy(aux.at[pl.ds(OFF_ROW + pl.multiple_of(ebase + b * SB, 8), SB)],
                              eb0.at[0].at[pl.ds(0, SB)], sem0).wait()
        pltpu.make_async_copy(aux.at[pl.ds(OFF_COL + pl.multiple_of(ebase + b * SB, 8), SB)],
                              eb0.at[1].at[pl.ds(0, SB)], sem0).wait()
        pltpu.make_async_copy(aux.at[pl.ds(OFF_VB + pl.multiple_of(ebase + b * SB, 8), SB)],
                              eb0.at[2].at[pl.ds(0, SB)], sem0).wait()
        load_sb(b + 1, eb1, sem1, SB)
        pos = process_sb(eb0, pos, SB // 16)
        pltpu.make_async_copy(aux.at[pl.ds(OFF_ROW + pl.multiple_of(ebase + (b + 1) * SB, 8), SB)],
                              eb1.at[0].at[pl.ds(0, SB)], sem1).wait()
        pltpu.make_async_copy(aux.at[pl.ds(OFF_COL + pl.multiple_of(ebase + (b + 1) * SB, 8), SB)],
                              eb1.at[1].at[pl.ds(0, SB)], sem1).wait()
        pltpu.make_async_copy(aux.at[pl.ds(OFF_VB + pl.multiple_of(ebase + (b + 1) * SB, 8), SB)],
                              eb1.at[2].at[pl.ds(0, SB)], sem1).wait()

        @pl.when(b + 2 < NSB)
        def _():
            load_sb(b + 2, eb0, sem0, SB)

        pos = process_sb(eb1, pos, SB // 16)
        return pos

    pos = lax.fori_loop(0, NSB // 2, pair, jnp.int32(0))

    ths = load_sb(NSB, eb0, sem0, TAIL)
    for h in ths:
        h.wait()
    pos = process_sb(eb0, pos, TAIL // 16)

    for i in range(8):
        scol[pl.ds(pos + i * 16, 16)] = zero16i
        sval[pl.ds(pos + i * 16, 16)] = zero16f
        srow[pl.ds(pos + i * 16, 16)] = zero16i

    @pl.when(pos > 0)
    def _():
        fire(0)

    plsc.subcore_barrier()

    @pl.loop(0, OUT_PER_SUB)
    def _(t):
        r = s * OUT_PER_SUB + t
        nidx = needbuf.at[pl.ds(pl.multiple_of(r * K, 8), K)]
        for i in range(8):
            n16 = needbuf[pl.ds(r * K + i * 16, 16)]
            ridx[pl.ds(i * 16, 16)] = plsc.load_gather(remap, [n16])
        h1 = pltpu.async_copy(ego_ref.at[nidx], ebuf, sem0)
        h2 = pltpu.async_copy(acc.at[ridx], abuf, sem1)
        h1.wait()
        h2.wait()

        third = jnp.float32(1.0 / 3.0)

        @pl.loop(0, K)
        def _(k):
            ebuf[k, pl.ds(0, 16)] = (
                ebuf[k, pl.ds(0, 16)] + 2.0 * abuf[k, pl.ds(0, 16)]) * third
            ebuf[k, pl.ds(16, 16)] = (
                ebuf[k, pl.ds(16, 16)] + 2.0 * abuf[k, pl.ds(16, 16)]) * third

        @pl.when(r < 8)
        def _():
            pltpu.sync_copy(
                ebuf, out_u.at[pl.ds(r * K, K), pl.ds(coff, HALF)])

        @pl.when((r >= 8) & (r < 16))
        def _():
            pltpu.sync_copy(
                ebuf, out_p.at[pl.ds((r - 8) * K, K), pl.ds(coff, HALF)])

        @pl.when(r >= 16)
        def _():
            pltpu.sync_copy(
                ebuf, out_n.at[pl.ds((r - 16) * K, K), pl.ds(coff, HALF)])


def _sc_kernel_body(ego_lo, ego_hi, aux,
                    out_u, out_p, out_n,
                    acc, remap, needbuf, eb0, eb1, scol, sval, srow,
                    frow, ridx, gbuf, gbuf2, frow2, zbuf, ebuf, abuf,
                    sem0, sem1, fsem, ssem):
    c = lax.axis_index("c")

    @pl.when(c == 0)
    def _():
        _half_pass(ego_lo, out_u, out_p, out_n, 0, aux, acc, remap, needbuf, eb0, eb1, scol, sval, srow,
                   frow, ridx, gbuf, gbuf2, frow2, zbuf, ebuf, abuf,
                   sem0, sem1, fsem, ssem)

    @pl.when(c == 1)
    def _():
        _half_pass(ego_hi, out_u, out_p, out_n, HALF, aux, acc, remap,
                   needbuf, eb0, eb1, scol, sval, srow,
                   frow, ridx, gbuf, gbuf2, frow2, zbuf, ebuf, abuf,
                   sem0, sem1, fsem, ssem)


def _compiler_params():
    cp = pltpu.CompilerParams(use_tc_tiling_on_sc=False)
    if "needs_layout_passes" in pltpu.CompilerParams.__dataclass_fields__:
        cp = dataclasses.replace(cp, needs_layout_passes=False)
    return cp


@jax.jit
def _run(ego_lo, ego_hi, aux):
    mesh = plsc.VectorSubcoreMesh(core_axis_name="c", subcore_axis_name="s")
    f32 = jnp.float32
    i32 = jnp.int32
    fn = pl.kernel(
        _sc_kernel_body,
        out_type=(
            jax.ShapeDtypeStruct((BATCH, EMB), f32),
            jax.ShapeDtypeStruct((BATCH, EMB), f32),
            jax.ShapeDtypeStruct((N_NEG, EMB), f32),
        ),
        mesh=mesh,
        compiler_params=_compiler_params(),
        scratch_types=[
            pltpu.VMEM_SHARED((ACC_ROWS, HALF), f32),
            pltpu.VMEM((N,), i32),
            pltpu.VMEM((NOUT,), i32),
            pltpu.VMEM((3, SB), i32),
            pltpu.VMEM((3, SB), i32),
            pltpu.VMEM((STG,), i32),
            pltpu.VMEM((STG,), f32),
            pltpu.VMEM((STG,), i32),
            pltpu.VMEM((K,), i32),
            pltpu.VMEM((K,), i32),
            pltpu.VMEM((K, HALF), f32),
            pltpu.VMEM((K, HALF), f32),
            pltpu.VMEM((K,), i32),
            pltpu.VMEM((ZROWS, HALF), f32),
            pltpu.VMEM((K, HALF), f32),
            pltpu.VMEM((K, HALF), f32),
            pltpu.SemaphoreType.DMA,
            pltpu.SemaphoreType.DMA,
            pltpu.SemaphoreType.DMA,
            pltpu.SemaphoreType.DMA,
        ],
    )
    return fn(ego_lo, ego_hi, aux)


def kernel(user_emb, item_emb, adj_val, users, pos_items, neg_items,
           adj_row, adj_col, mask, norm_adj):
    ego = jnp.concatenate([user_emb, item_emb], axis=0)
    ego_lo = ego[:, :HALF]
    ego_hi = ego[:, HALF:]
    vbits = lax.bitcast_convert_type(adj_val, jnp.int32)
    need = jnp.concatenate(
        [users, pos_items + N_USER, neg_items + N_USER]).astype(jnp.int32)
    aux = jnp.concatenate(
        [adj_row, adj_col, vbits, jnp.full((N,), -1, jnp.int32), need])
    return _run(ego_lo, ego_hi, aux)


# speedup vs baseline: 1.2394x; 1.2394x over previous
import dataclasses

import jax
import jax.numpy as jnp
from jax import lax
from jax.experimental import pallas as pl
from jax.experimental.pallas import tpu as pltpu
from jax.experimental.pallas import tpu_sc as plsc

N_USER = 25000
N_ITEM = 25000
N = N_USER + N_ITEM
E = 800000
EMB = 64
HALF = EMB // 2
BATCH = 1024
N_NEG = 4096
NOUT = BATCH + BATCH + N_NEG

NS = 16
EPW = E // NS
SB = 4096
NSB = 12
TAIL = EPW - NSB * SB
K = 128
STG = SB + 2 * K
ACC_ROWS = NOUT
ZROWS = 48
ZREP = (ACC_ROWS // NS) // ZROWS
OUT_PER_SUB = (NOUT // K) // NS


def _half_pass(ego_ref, out_u, out_p, out_n, coff, rowf, colf, vbits, need,
               negs, acc, remap, needbuf, eb0, eb1, scol, sval, srow, frow,
               ridx, gbuf, gbuf2, frow2, zbuf, ebuf, abuf, sem0, sem1,
               fsem, ssem):
    s = lax.axis_index("s")
    zero16i = jnp.zeros((16,), jnp.int32)
    zero16f = jnp.zeros((16,), jnp.float32)
    neg16i = jnp.full((16,), -1, jnp.int32)
    iota16 = lax.iota(jnp.int32, 16)
    ebase = s * EPW
    pltpu.async_copy(rowf.at[pl.ds(ebase, SB)], eb0.at[0], sem0)
    pltpu.async_copy(colf.at[pl.ds(ebase, SB)], eb0.at[1], sem0)
    pltpu.async_copy(vbits.at[pl.ds(ebase, SB)], eb0.at[2], sem0)

    hneg = pltpu.async_copy(negs, remap, sem1)
    pltpu.sync_copy(need, needbuf)

    hneg.wait()

    @pl.loop(0, NOUT // 16)
    def _(i):
        idx16 = needbuf[pl.ds(i * 16, 16)]
        plsc.store_scatter(remap, [idx16], i * 16 + iota16)

    @pl.loop(0, ZROWS)
    def _(i):
        zbuf[i, pl.ds(0, 16)] = zero16f
        zbuf[i, pl.ds(16, 16)] = zero16f

    @pl.loop(0, ZREP)
    def _(j):
        pltpu.sync_copy(
            zbuf, acc.at[pl.ds(s * (ACC_ROWS // NS) + j * ZROWS, ZROWS)])

    plsc.subcore_barrier()

    def load_sb(b, buf, sem, size):
        base = pl.multiple_of(ebase + b * SB, 8)
        hr = pltpu.async_copy(rowf.at[pl.ds(base, size)],
                              buf.at[0].at[pl.ds(0, size)], sem)
        hc = pltpu.async_copy(colf.at[pl.ds(base, size)],
                              buf.at[1].at[pl.ds(0, size)], sem)
        hv = pltpu.async_copy(vbits.at[pl.ds(base, size)],
                              buf.at[2].at[pl.ds(0, size)], sem)
        return (hr, hc, hv)

    def fire_issue(fb, frowx, gbufx):
        fb = pl.multiple_of(fb, 8)
        for i in range(8):
            frowx[pl.ds(i * 16, 16)] = srow[pl.ds(fb + i * 16, 16)]
        return pltpu.async_copy(ego_ref.at[scol.at[pl.ds(fb, K)]], gbufx,
                                fsem)

    def fire_scale(fb, gbufx):
        fb = pl.multiple_of(fb, 8)

        @pl.loop(0, K, step=16)
        def _(c0):
            vals = sval[pl.ds(fb + c0, 16)]
            for i in range(16):
                vi = vals.at[jnp.full((16,), i, jnp.int32)].get(
                    mode="promise_in_bounds")
                k = c0 + i
                gbufx[k, pl.ds(0, 16)] = gbufx[k, pl.ds(0, 16)] * vi
                gbufx[k, pl.ds(16, 16)] = gbufx[k, pl.ds(16, 16)] * vi

    def fire(fb):
        h = fire_issue(fb, frow, gbuf)
        h.wait()
        fire_scale(fb, gbuf)
        pltpu.sync_copy(gbuf, acc.at[frow], add=True)

    def process_sb(buf, pos, ngroups):
        def grp(g, pos):
            r16 = buf[0, pl.ds(g * 16, 16)]
            m16 = plsc.load_gather(remap, [r16])
            pred = m16 >= 0
            c16 = buf[1, pl.ds(g * 16, 16)]
            v16 = plsc.bitcast(buf[2, pl.ds(g * 16, 16)], jnp.float32)
            plsc.store_compressed(scol.at[pl.ds(pos, 16)], c16, mask=pred)
            plsc.store_compressed(sval.at[pl.ds(pos, 16)], v16, mask=pred)
            plsc.store_compressed(srow.at[pl.ds(pos, 16)], m16, mask=pred)
            cnt = plsc.all_reduce_population_count(pred)
            return pos + cnt[0]

        pos = lax.fori_loop(0, ngroups, grp, pos, unroll=4)

        def fire_cond(st):
            j, p = st
            return j + K <= p

        def fire_body(st):
            j, p = st
            two = j + 2 * K <= p
            ha = fire_issue(j, frow, gbuf)

            @pl.when(two)
            def _():
                fire_issue(j + K, frow2, gbuf2)

            ha.wait()
            fire_scale(j, gbuf)
            hsa = pltpu.async_copy(gbuf, acc.at[frow], ssem, add=True)

            @pl.when(two)
            def _():
                fbb = pl.multiple_of(j + K, 8)
                pltpu.make_async_copy(
                    ego_ref.at[scol.at[pl.ds(fbb, K)]], gbuf2, fsem).wait()
                fire_scale(j + K, gbuf2)
                pltpu.async_copy(gbuf2, acc.at[frow2], ssem, add=True)

            hsa.wait()

            @pl.when(two)
            def _():
                pltpu.make_async_copy(gbuf2, acc.at[frow2], ssem).wait()

            return (j + K + jnp.where(two, K, 0).astype(jnp.int32), p)

        fb, pos = lax.while_loop(fire_cond, fire_body, (jnp.int32(0), pos))
        for i in range(8):
            scol[pl.ds(i * 16, 16)] = scol[pl.ds(fb + i * 16, 16)]
            sval[pl.ds(i * 16, 16)] = sval[pl.ds(fb + i * 16, 16)]
            srow[pl.ds(i * 16, 16)] = srow[pl.ds(fb + i * 16, 16)]
        return pos - fb

    def pair(q, pos):
        b = q * 2
        pltpu.make_async_copy(rowf.at[pl.ds(pl.multiple_of(ebase + b * SB, 8), SB)],
                              eb0.at[0].at[pl.ds(0, SB)], sem0).wait()
        pltpu.make_async_copy(colf.at[pl.ds(pl.multiple_of(ebase + b * SB, 8), SB)],
                              eb0.at[1].at[pl.ds(0, SB)], sem0).wait()
        pltpu.make_async_copy(vbits.at[pl.ds(pl.multiple_of(ebase + b * SB, 8), SB)],
                              eb0.at[2].at[pl.ds(0, SB)], sem0).wait()
        load_sb(b + 1, eb1, sem1, SB)
        pos = process_sb(eb0, pos, SB // 16)
        pltpu.make_async_copy(rowf.at[pl.ds(pl.multiple_of(ebase + (b + 1) * SB, 8), SB)],
                              eb1.at[0].at[pl.ds(0, SB)], sem1).wait()
        pltpu.make_async_copy(colf.at[pl.ds(pl.multiple_of(ebase + (b + 1) * SB, 8), SB)],
                              eb1.at[1].at[pl.ds(0, SB)], sem1).wait()
        pltpu.make_async_copy(vbits.at[pl.ds(pl.multiple_of(ebase + (b + 1) * SB, 8), SB)],
                              eb1.at[2].at[pl.ds(0, SB)], sem1).wait()

        @pl.when(b + 2 < NSB)
        def _():
            load_sb(b + 2, eb0, sem0, SB)

        pos = process_sb(eb1, pos, SB // 16)
        return pos

    pos = lax.fori_loop(0, NSB // 2, pair, jnp.int32(0))

    ths = load_sb(NSB, eb0, sem0, TAIL)
    for h in ths:
        h.wait()
    pos = process_sb(eb0, pos, TAIL // 16)

    for i in range(8):
        scol[pl.ds(pos + i * 16, 16)] = zero16i
        sval[pl.ds(pos + i * 16, 16)] = zero16f
        srow[pl.ds(pos + i * 16, 16)] = zero16i

    @pl.when(pos > 0)
    def _():
        fire(0)

    plsc.subcore_barrier()

    @pl.loop(0, OUT_PER_SUB)
    def _(t):
        r = s * OUT_PER_SUB + t
        nidx = needbuf.at[pl.ds(pl.multiple_of(r * K, 8), K)]
        for i in range(8):
            n16 = needbuf[pl.ds(r * K + i * 16, 16)]
            ridx[pl.ds(i * 16, 16)] = plsc.load_gather(remap, [n16])
        h1 = pltpu.async_copy(ego_ref.at[nidx], ebuf, sem0)
        h2 = pltpu.async_copy(acc.at[ridx], abuf, sem1)
        h1.wait()
        h2.wait()

        third = jnp.float32(1.0 / 3.0)

        @pl.loop(0, K)
        def _(k):
            ebuf[k, pl.ds(0, 16)] = (
                ebuf[k, pl.ds(0, 16)] + 2.0 * abuf[k, pl.ds(0, 16)]) * third
            ebuf[k, pl.ds(16, 16)] = (
                ebuf[k, pl.ds(16, 16)] + 2.0 * abuf[k, pl.ds(16, 16)]) * third

        @pl.when(r < 8)
        def _():
            pltpu.sync_copy(
                ebuf, out_u.at[pl.ds(r * K, K), pl.ds(coff, HALF)])

        @pl.when((r >= 8) & (r < 16))
        def _():
            pltpu.sync_copy(
                ebuf, out_p.at[pl.ds((r - 8) * K, K), pl.ds(coff, HALF)])

        @pl.when(r >= 16)
        def _():
            pltpu.sync_copy(
                ebuf, out_n.at[pl.ds((r - 16) * K, K), pl.ds(coff, HALF)])


def _sc_kernel_body(ego_lo, ego_hi, rowf, colf, vbits, need, negs,
                    out_u, out_p, out_n,
                    acc, remap, needbuf, eb0, eb1, scol, sval, srow,
                    frow, ridx, gbuf, gbuf2, frow2, zbuf, ebuf, abuf,
                    sem0, sem1, fsem, ssem):
    c = lax.axis_index("c")

    @pl.when(c == 0)
    def _():
        _half_pass(ego_lo, out_u, out_p, out_n, 0, rowf, colf, vbits, need,
                   negs, acc, remap, needbuf, eb0, eb1, scol, sval, srow,
                   frow, ridx, gbuf, gbuf2, frow2, zbuf, ebuf, abuf,
                   sem0, sem1, fsem, ssem)

    @pl.when(c == 1)
    def _():
        _half_pass(ego_hi, out_u, out_p, out_n, HALF, rowf, colf, vbits, need,
                   negs, acc, remap, needbuf, eb0, eb1, scol, sval, srow,
                   frow, ridx, gbuf, gbuf2, frow2, zbuf, ebuf, abuf,
                   sem0, sem1, fsem, ssem)


def _compiler_params():
    cp = pltpu.CompilerParams(use_tc_tiling_on_sc=False)
    if "needs_layout_passes" in pltpu.CompilerParams.__dataclass_fields__:
        cp = dataclasses.replace(cp, needs_layout_passes=False)
    return cp


@jax.jit
def _run(ego_lo, ego_hi, rowf, colf, vbits, need, negs):
    mesh = plsc.VectorSubcoreMesh(core_axis_name="c", subcore_axis_name="s")
    f32 = jnp.float32
    i32 = jnp.int32
    fn = pl.kernel(
        _sc_kernel_body,
        out_type=(
            jax.ShapeDtypeStruct((BATCH, EMB), f32),
            jax.ShapeDtypeStruct((BATCH, EMB), f32),
            jax.ShapeDtypeStruct((N_NEG, EMB), f32),
        ),
        mesh=mesh,
        compiler_params=_compiler_params(),
        scratch_types=[
            pltpu.VMEM_SHARED((ACC_ROWS, HALF), f32),
            pltpu.VMEM((N,), i32),
            pltpu.VMEM((NOUT,), i32),
            pltpu.VMEM((3, SB), i32),
            pltpu.VMEM((3, SB), i32),
            pltpu.VMEM((STG,), i32),
            pltpu.VMEM((STG,), f32),
            pltpu.VMEM((STG,), i32),
            pltpu.VMEM((K,), i32),
            pltpu.VMEM((K,), i32),
            pltpu.VMEM((K, HALF), f32),
            pltpu.VMEM((K, HALF), f32),
            pltpu.VMEM((K,), i32),
            pltpu.VMEM((ZROWS, HALF), f32),
            pltpu.VMEM((K, HALF), f32),
            pltpu.VMEM((K, HALF), f32),
            pltpu.SemaphoreType.DMA,
            pltpu.SemaphoreType.DMA,
            pltpu.SemaphoreType.DMA,
            pltpu.SemaphoreType.DMA,
        ],
    )
    return fn(ego_lo, ego_hi, rowf, colf, vbits, need, negs)


def kernel(user_emb, item_emb, adj_val, users, pos_items, neg_items,
           adj_row, adj_col, mask, norm_adj):
    ego = jnp.concatenate([user_emb, item_emb], axis=0)
    ego_lo = ego[:, :HALF]
    ego_hi = ego[:, HALF:]
    vbits = lax.bitcast_convert_type(adj_val, jnp.int32)
    need = jnp.concatenate(
        [users, pos_items + N_USER, neg_items + N_USER]).astype(jnp.int32)
    negs = jnp.full((N,), -1, jnp.int32)
    return _run(ego_lo, ego_hi, adj_row, adj_col, vbits, need, negs)


# speedup vs baseline: 1.4545x; 1.1736x over previous
import dataclasses

import jax
import jax.numpy as jnp
from jax import lax
from jax.experimental import pallas as pl
from jax.experimental.pallas import tpu as pltpu
from jax.experimental.pallas import tpu_sc as plsc

N_USER = 25000
N_ITEM = 25000
N = N_USER + N_ITEM
E = 800000
EMB = 64
HALF = EMB // 2
BATCH = 1024
N_NEG = 4096
NOUT = BATCH + BATCH + N_NEG

NS = 16
EPW = E // NS
SB = 4096
NSB = 12
TAIL = EPW - NSB * SB
K = 128
STG = SB + 2 * K
ACC_ROWS = NOUT
ZROWS = 48
ZREP = (ACC_ROWS // NS) // ZROWS
OUT_PER_SUB = (NOUT // K) // NS


def _half_pass(ego_ref, out_u, out_p, out_n, coff, rowf, colf, vbits, need,
               negs, acc, remap, needbuf, eb0, eb1, scol, sval, srow, frow,
               ridx, gbuf, gbuf2, frow2, zbuf, ebuf, abuf, sem0, sem1,
               fsem, ssem):
    s = lax.axis_index("s")
    zero16i = jnp.zeros((16,), jnp.int32)
    zero16f = jnp.zeros((16,), jnp.float32)
    neg16i = jnp.full((16,), -1, jnp.int32)
    iota16 = lax.iota(jnp.int32, 16)
    ebase = s * EPW
    pltpu.async_copy(rowf.at[pl.ds(ebase, SB)], eb0.at[0], sem0)
    pltpu.async_copy(colf.at[pl.ds(ebase, SB)], eb0.at[1], sem0)
    pltpu.async_copy(vbits.at[pl.ds(ebase, SB)], eb0.at[2], sem0)

    hneg = pltpu.async_copy(negs, remap, sem1)
    pltpu.sync_copy(need, needbuf)

    hneg.wait()

    @pl.loop(0, NOUT // 16)
    def _(i):
        idx16 = needbuf[pl.ds(i * 16, 16)]
        plsc.store_scatter(remap, [idx16], i * 16 + iota16)

    @pl.loop(0, ZROWS)
    def _(i):
        zbuf[i, pl.ds(0, 16)] = zero16f
        zbuf[i, pl.ds(16, 16)] = zero16f

    @pl.loop(0, ZREP)
    def _(j):
        pltpu.sync_copy(
            zbuf, acc.at[pl.ds(s * (ACC_ROWS // NS) + j * ZROWS, ZROWS)])

    plsc.subcore_barrier()

    def load_sb(b, buf, sem, size):
        base = pl.multiple_of(ebase + b * SB, 8)
        hr = pltpu.async_copy(rowf.at[pl.ds(base, size)],
                              buf.at[0].at[pl.ds(0, size)], sem)
        hc = pltpu.async_copy(colf.at[pl.ds(base, size)],
                              buf.at[1].at[pl.ds(0, size)], sem)
        hv = pltpu.async_copy(vbits.at[pl.ds(base, size)],
                              buf.at[2].at[pl.ds(0, size)], sem)
        return (hr, hc, hv)

    def fire_issue(fb, frowx, gbufx):
        fb = pl.multiple_of(fb, 8)
        for i in range(8):
            frowx[pl.ds(i * 16, 16)] = srow[pl.ds(fb + i * 16, 16)]
        return pltpu.async_copy(ego_ref.at[scol.at[pl.ds(fb, K)]], gbufx,
                                fsem)

    def fire_scale(fb, gbufx):
        fb = pl.multiple_of(fb, 8)

        @pl.loop(0, K, step=16)
        def _(c0):
            vals = sval[pl.ds(fb + c0, 16)]
            for i in range(16):
                vi = vals.at[jnp.full((16,), i, jnp.int32)].get(
                    mode="promise_in_bounds")
                k = c0 + i
                gbufx[k, pl.ds(0, 16)] = gbufx[k, pl.ds(0, 16)] * vi
                gbufx[k, pl.ds(16, 16)] = gbufx[k, pl.ds(16, 16)] * vi

    def fire(fb):
        h = fire_issue(fb, frow, gbuf)
        h.wait()
        fire_scale(fb, gbuf)
        pltpu.sync_copy(gbuf, acc.at[frow], add=True)

    def process_sb(buf, pos, ngroups):
        def grp(g, pos):
            r16 = buf[0, pl.ds(g * 16, 16)]
            m16 = plsc.load_gather(remap, [r16])
            pred = m16 >= 0
            c16 = buf[1, pl.ds(g * 16, 16)]
            v16 = plsc.bitcast(buf[2, pl.ds(g * 16, 16)], jnp.float32)
            plsc.store_compressed(scol.at[pl.ds(pos, 16)], c16, mask=pred)
            plsc.store_compressed(sval.at[pl.ds(pos, 16)], v16, mask=pred)
            plsc.store_compressed(srow.at[pl.ds(pos, 16)], m16, mask=pred)
            cnt = plsc.all_reduce_population_count(pred)
            return pos + cnt[0]

        @plsc.parallel_loop(0, ngroups, unroll=4, carry=pos)
        def pos(g, pos):
            return grp(g, pos)

        def fire_cond(st):
            j, p = st
            return j + K <= p

        def fire_body(st):
            j, p = st
            two = j + 2 * K <= p
            ha = fire_issue(j, frow, gbuf)

            @pl.when(two)
            def _():
                fire_issue(j + K, frow2, gbuf2)

            ha.wait()
            fire_scale(j, gbuf)
            hsa = pltpu.async_copy(gbuf, acc.at[frow], ssem, add=True)

            @pl.when(two)
            def _():
                fbb = pl.multiple_of(j + K, 8)
                pltpu.make_async_copy(
                    ego_ref.at[scol.at[pl.ds(fbb, K)]], gbuf2, fsem).wait()
                fire_scale(j + K, gbuf2)
                pltpu.async_copy(gbuf2, acc.at[frow2], ssem, add=True)

            hsa.wait()

            @pl.when(two)
            def _():
                pltpu.make_async_copy(gbuf2, acc.at[frow2], ssem).wait()

            return (j + K + jnp.where(two, K, 0).astype(jnp.int32), p)

        fb, pos = lax.while_loop(fire_cond, fire_body, (jnp.int32(0), pos))
        for i in range(8):
            scol[pl.ds(i * 16, 16)] = scol[pl.ds(fb + i * 16, 16)]
            sval[pl.ds(i * 16, 16)] = sval[pl.ds(fb + i * 16, 16)]
            srow[pl.ds(i * 16, 16)] = srow[pl.ds(fb + i * 16, 16)]
        return pos - fb

    def pair(q, pos):
        b = q * 2
        pltpu.make_async_copy(rowf.at[pl.ds(pl.multiple_of(ebase + b * SB, 8), SB)],
                              eb0.at[0].at[pl.ds(0, SB)], sem0).wait()
        pltpu.make_async_copy(colf.at[pl.ds(pl.multiple_of(ebase + b * SB, 8), SB)],
                              eb0.at[1].at[pl.ds(0, SB)], sem0).wait()
        pltpu.make_async_copy(vbits.at[pl.ds(pl.multiple_of(ebase + b * SB, 8), SB)],
                              eb0.at[2].at[pl.ds(0, SB)], sem0).wait()
        load_sb(b + 1, eb1, sem1, SB)
        pos = process_sb(eb0, pos, SB // 16)
        pltpu.make_async_copy(rowf.at[pl.ds(pl.multiple_of(ebase + (b + 1) * SB, 8), SB)],
                              eb1.at[0].at[pl.ds(0, SB)], sem1).wait()
        pltpu.make_async_copy(colf.at[pl.ds(pl.multiple_of(ebase + (b + 1) * SB, 8), SB)],
                              eb1.at[1].at[pl.ds(0, SB)], sem1).wait()
        pltpu.make_async_copy(vbits.at[pl.ds(pl.multiple_of(ebase + (b + 1) * SB, 8), SB)],
                              eb1.at[2].at[pl.ds(0, SB)], sem1).wait()

        @pl.when(b + 2 < NSB)
        def _():
            load_sb(b + 2, eb0, sem0, SB)

        pos = process_sb(eb1, pos, SB // 16)
        return pos

    pos = lax.fori_loop(0, NSB // 2, pair, jnp.int32(0))

    ths = load_sb(NSB, eb0, sem0, TAIL)
    for h in ths:
        h.wait()
    pos = process_sb(eb0, pos, TAIL // 16)

    for i in range(8):
        scol[pl.ds(pos + i * 16, 16)] = zero16i
        sval[pl.ds(pos + i * 16, 16)] = zero16f
        srow[pl.ds(pos + i * 16, 16)] = zero16i

    @pl.when(pos > 0)
    def _():
        fire(0)

    plsc.subcore_barrier()

    @pl.loop(0, OUT_PER_SUB)
    def _(t):
        r = s * OUT_PER_SUB + t
        nidx = needbuf.at[pl.ds(pl.multiple_of(r * K, 8), K)]
        for i in range(8):
            n16 = needbuf[pl.ds(r * K + i * 16, 16)]
            ridx[pl.ds(i * 16, 16)] = plsc.load_gather(remap, [n16])
        h1 = pltpu.async_copy(ego_ref.at[nidx], ebuf, sem0)
        h2 = pltpu.async_copy(acc.at[ridx], abuf, sem1)
        h1.wait()
        h2.wait()

        third = jnp.float32(1.0 / 3.0)

        @pl.loop(0, K)
        def _(k):
            ebuf[k, pl.ds(0, 16)] = (
                ebuf[k, pl.ds(0, 16)] + 2.0 * abuf[k, pl.ds(0, 16)]) * third
            ebuf[k, pl.ds(16, 16)] = (
                ebuf[k, pl.ds(16, 16)] + 2.0 * abuf[k, pl.ds(16, 16)]) * third

        @pl.when(r < 8)
        def _():
            pltpu.sync_copy(
                ebuf, out_u.at[pl.ds(r * K, K), pl.ds(coff, HALF)])

        @pl.when((r >= 8) & (r < 16))
        def _():
            pltpu.sync_copy(
                ebuf, out_p.at[pl.ds((r - 8) * K, K), pl.ds(coff, HALF)])

        @pl.when(r >= 16)
        def _():
            pltpu.sync_copy(
                ebuf, out_n.at[pl.ds((r - 16) * K, K), pl.ds(coff, HALF)])


def _sc_kernel_body(ego_lo, ego_hi, rowf, colf, vbits, need, negs,
                    out_u, out_p, out_n,
                    acc, remap, needbuf, eb0, eb1, scol, sval, srow,
                    frow, ridx, gbuf, gbuf2, frow2, zbuf, ebuf, abuf,
                    sem0, sem1, fsem, ssem):
    c = lax.axis_index("c")

    @pl.when(c == 0)
    def _():
        _half_pass(ego_lo, out_u, out_p, out_n, 0, rowf, colf, vbits, need,
                   negs, acc, remap, needbuf, eb0, eb1, scol, sval, srow,
                   frow, ridx, gbuf, gbuf2, frow2, zbuf, ebuf, abuf,
                   sem0, sem1, fsem, ssem)

    @pl.when(c == 1)
    def _():
        _half_pass(ego_hi, out_u, out_p, out_n, HALF, rowf, colf, vbits, need,
                   negs, acc, remap, needbuf, eb0, eb1, scol, sval, srow,
                   frow, ridx, gbuf, gbuf2, frow2, zbuf, ebuf, abuf,
                   sem0, sem1, fsem, ssem)


def _compiler_params():
    cp = pltpu.CompilerParams(use_tc_tiling_on_sc=False)
    if "needs_layout_passes" in pltpu.CompilerParams.__dataclass_fields__:
        cp = dataclasses.replace(cp, needs_layout_passes=False)
    return cp


@jax.jit
def _run(ego_lo, ego_hi, rowf, colf, vbits, need, negs):
    mesh = plsc.VectorSubcoreMesh(core_axis_name="c", subcore_axis_name="s")
    f32 = jnp.float32
    i32 = jnp.int32
    fn = pl.kernel(
        _sc_kernel_body,
        out_type=(
            jax.ShapeDtypeStruct((BATCH, EMB), f32),
            jax.ShapeDtypeStruct((BATCH, EMB), f32),
            jax.ShapeDtypeStruct((N_NEG, EMB), f32),
        ),
        mesh=mesh,
        compiler_params=_compiler_params(),
        scratch_types=[
            pltpu.VMEM_SHARED((ACC_ROWS, HALF), f32),
            pltpu.VMEM((N,), i32),
            pltpu.VMEM((NOUT,), i32),
            pltpu.VMEM((3, SB), i32),
            pltpu.VMEM((3, SB), i32),
            pltpu.VMEM((STG,), i32),
            pltpu.VMEM((STG,), f32),
            pltpu.VMEM((STG,), i32),
            pltpu.VMEM((K,), i32),
            pltpu.VMEM((K,), i32),
            pltpu.VMEM((K, HALF), f32),
            pltpu.VMEM((K, HALF), f32),
            pltpu.VMEM((K,), i32),
            pltpu.VMEM((ZROWS, HALF), f32),
            pltpu.VMEM((K, HALF), f32),
            pltpu.VMEM((K, HALF), f32),
            pltpu.SemaphoreType.DMA,
            pltpu.SemaphoreType.DMA,
            pltpu.SemaphoreType.DMA,
            pltpu.SemaphoreType.DMA,
        ],
    )
    return fn(ego_lo, ego_hi, rowf, colf, vbits, need, negs)


def kernel(user_emb, item_emb, adj_val, users, pos_items, neg_items,
           adj_row, adj_col, mask, norm_adj):
    ego = jnp.concatenate([user_emb, item_emb], axis=0)
    ego_lo = ego[:, :HALF]
    ego_hi = ego[:, HALF:]
    vbits = lax.bitcast_convert_type(adj_val, jnp.int32)
    need = jnp.concatenate(
        [users, pos_items + N_USER, neg_items + N_USER]).astype(jnp.int32)
    negs = jnp.full((N,), -1, jnp.int32)
    return _run(ego_lo, ego_hi, adj_row, adj_col, vbits, need, negs)
